# baseline (speedup 1.0000x reference)
import jax
import jax.numpy as jnp
from jax.experimental import pallas as pl

B = 1024
P = 10
LOUT = 21
D = 1024

BB = 128
RB = 8
NJ = 3


def _fill_body(vtok_ref, end_ref, rep_ref, out_ref):
    j = pl.program_id(1)
    rep_row = rep_ref[...][None, :, :]

    @pl.when(j == 0)
    def _rows_0_7():
        out_ref[...] = vtok_ref[:, 0:RB, :]

    @pl.when(j == 1)
    def _rows_8_15():
        out_ref[...] = jnp.concatenate(
            [vtok_ref[:, RB:P, :],
             jnp.broadcast_to(end_ref[...][None, :, :], (BB, 1, D)),
             jnp.broadcast_to(rep_row, (BB, 5, D))], axis=1)

    @pl.when(j == 2)
    def _rows_16_20():
        out_ref[...] = jnp.broadcast_to(rep_row, (BB, RB, D))


def kernel(categories, vtok, end, zero):
    rep = jnp.where(categories[0, 11] == 0, zero, end)
    return pl.pallas_call(
        _fill_body,
        grid=(B // BB, NJ),
        in_specs=[
            pl.BlockSpec((BB, P, D), lambda i, j: (i, 0, 0)),
            pl.BlockSpec((1, D), lambda i, j: (0, 0)),
            pl.BlockSpec((1, D), lambda i, j: (0, 0)),
        ],
        out_specs=pl.BlockSpec((BB, RB, D), lambda i, j: (i, j, 0)),
        out_shape=jax.ShapeDtypeStruct((B, LOUT, D), jnp.float32),
    )(vtok, end, rep)
